# baseline (speedup 1.0000x reference)
import functools

import jax
import jax.numpy as jnp
from jax import lax
from jax.experimental import pallas as pl
from jax.experimental.pallas import tpu as pltpu
from jax.experimental.pallas import tpu_sc as plsc

F = 26
V = 100000
D = 32
B = 16384
NUM_NUMERIC = 13
H1 = 256
H2 = 64

NGRP = 7
GB = 128
CH = 1024
NG = CH // GB
TOTR = NGRP * B * 4


def _sc_gather(cat_grouped, tables_flat):
    info = plsc.get_sparse_core_info()
    NC, NS = info.num_cores, info.num_subcores
    NW = NC * NS
    per_w = TOTR // NW
    nch = per_w // CH
    rows_per_grp = B * 4

    @functools.partial(
        pl.kernel,
        mesh=plsc.VectorSubcoreMesh(core_axis_name="c", subcore_axis_name="s"),
        out_type=jax.ShapeDtypeStruct((TOTR, D), jnp.float32),
        scratch_types=[
            pltpu.VMEM((4, CH // 4), jnp.int32),
            pltpu.VMEM((NG, GB), jnp.int32),
            pltpu.VMEM((CH, D), jnp.float32),
            pltpu.SemaphoreType.DMA,
        ],
        compiler_params=pltpu.CompilerParams(
            use_tc_tiling_on_sc=False, needs_layout_passes=False
        ),
    )
    def gather_k(cat_hbm, tab_hbm, out_hbm, cat_v, idx_v, rows_v, sem):
        wid = lax.axis_index("s") * NC + lax.axis_index("c")
        lane = lax.iota(jnp.int32, 16)
        jvec = lane % 4
        bvec = lane // 4

        @pl.loop(0, nch)
        def _chunk(c):
            base = pl.multiple_of(wid * per_w + c * CH, CH)
            g = base // rows_per_grp
            b0 = pl.multiple_of((base // 4) % B, CH // 4)
            pltpu.sync_copy(cat_hbm.at[g, :, pl.ds(b0, CH // 4)], cat_v)

            fbase = g * 4
            foff = (fbase + jvec) * V
            valid = (fbase + jvec) < F

            @pl.loop(0, CH // 16)
            def _vec(i):
                raw = plsc.load_gather(cat_v, [jvec, i * 4 + bvec])
                idx = jnp.where(valid, raw + foff, 0)
                idx_v[i // 8, pl.ds((i % 8) * 16, 16)] = idx

            copies = [
                pltpu.async_copy(
                    tab_hbm.at[idx_v.at[r]],
                    rows_v.at[pl.ds(r * GB, GB)],
                    sem,
                )
                for r in range(NG)
            ]
            for cp in copies:
                cp.wait()
            pltpu.sync_copy(rows_v, out_hbm.at[pl.ds(base, CH)])

    return gather_k(cat_grouped, tables_flat)


def _tc_mlp(x3, num_pad, w1a3, w1b, b1, w2, b2):
    bb = 512
    grid = (B // bb,)

    def body(x_ref, n_ref, w1a_ref, w1b_ref, b1_ref, w2_ref, b2_ref, o_ref):
        h = jnp.dot(n_ref[...], w1b_ref[...], preferred_element_type=jnp.float32)
        for g in range(NGRP):
            h += jnp.dot(x_ref[g], w1a_ref[g],
                         preferred_element_type=jnp.float32)
        h = jnp.maximum(h + b1_ref[...], 0.0)
        o = jnp.dot(h, w2_ref[...], preferred_element_type=jnp.float32) + b2_ref[...]
        o_ref[...] = jnp.maximum(o, 0.0)

    return pl.pallas_call(
        body,
        grid=grid,
        in_specs=[
            pl.BlockSpec((NGRP, bb, 128), lambda i: (0, i, 0)),
            pl.BlockSpec((bb, 16), lambda i: (i, 0)),
            pl.BlockSpec((NGRP, 128, H1), lambda i: (0, 0, 0)),
            pl.BlockSpec((16, H1), lambda i: (0, 0)),
            pl.BlockSpec((1, H1), lambda i: (0, 0)),
            pl.BlockSpec((H1, H2), lambda i: (0, 0)),
            pl.BlockSpec((1, H2), lambda i: (0, 0)),
        ],
        out_specs=pl.BlockSpec((bb, H2), lambda i: (i, 0)),
        out_shape=jax.ShapeDtypeStruct((B, H2), jnp.float32),
        compiler_params=pltpu.CompilerParams(
            dimension_semantics=("arbitrary",),
        ),
    )(x3, num_pad, w1a3, w1b, b1, w2, b2)


def kernel(cat_input, num_input, tables, W1, b1, W2, b2):
    cat_t = jnp.pad(cat_input.T, ((0, 4 * NGRP - F), (0, 0)))
    cat_grouped = cat_t.reshape(NGRP, 4, B)
    tables_flat = tables.reshape(F * V, D)

    embeds = _sc_gather(cat_grouped, tables_flat)
    x3 = embeds.reshape(NGRP, B, 4 * D)

    num_pad = jnp.pad(num_input, ((0, 0), (0, 16 - NUM_NUMERIC)))
    w1a3 = jnp.pad(W1[: F * D], ((0, 4 * NGRP * D - F * D), (0, 0)))
    w1a3 = w1a3.reshape(NGRP, 4 * D, H1)
    w1b = jnp.pad(W1[F * D :], ((0, 16 - NUM_NUMERIC), (0, 0)))
    return _tc_mlp(x3, num_pad, w1a3, w1b,
                   b1.reshape(1, H1), W2, b2.reshape(1, H2))
